# Initial kernel scaffold; baseline (speedup 1.0000x reference)
import jax
import jax.numpy as jnp
from jax.experimental import pallas as pl


def kernel(predictions, labels, upper_region):
    raise NotImplementedError("write your pallas kernel here")


# speedup vs baseline: 3.2516x; 3.2516x over previous
import jax
import jax.numpy as jnp
from jax.experimental import pallas as pl
from jax.experimental.pallas import tpu as pltpu

_B, _C, _H, _W = 8, 2, 512, 512
_CHUNKS = 4
_RC = _H // _CHUNKS


def _loss_kernel(preds_ref, labels_ref, upper_ref, out_ref, acc_ref):
    b = pl.program_id(0)
    c = pl.program_id(1)

    l = labels_ref[0]
    u = upper_ref[0]
    p0 = preds_ref[0, 0]
    p1 = preds_ref[0, 1]

    neural = l == 255
    nonneural = (l == 0) & (u == 255)
    mask = neural | nonneural

    mx = jnp.maximum(p0, p1)
    mn = jnp.minimum(p0, p1)
    logz = mx + jnp.log1p(jnp.exp(mn - mx))
    ll = jnp.where(neural, p1, p0)
    val = jnp.where(mask, logz - ll, 0.0)

    s_part = jnp.sum(val)
    n1_part = jnp.sum(neural.astype(jnp.float32))
    n2_part = jnp.sum(nonneural.astype(jnp.float32))

    @pl.when(c == 0)
    def _reset_batch():
        acc_ref[0] = 0.0
        acc_ref[1] = 0.0
        acc_ref[2] = 0.0

    @pl.when((b == 0) & (c == 0))
    def _reset_total():
        acc_ref[3] = 0.0
        acc_ref[4] = 0.0

    acc_ref[0] += s_part
    acc_ref[1] += n1_part
    acc_ref[2] += n2_part

    @pl.when(c == _CHUNKS - 1)
    def _finish_batch():
        s = acc_ref[0]
        n1 = acc_ref[1]
        n2 = acc_ref[2]
        ok = (n1 > 0.0) & (n2 > 0.0)
        denom = n1 + n2
        contrib = s / jnp.where(denom > 0.0, denom, 1.0)
        acc_ref[3] += jnp.where(ok, contrib, 0.0)
        acc_ref[4] += jnp.where(ok, 1.0, 0.0)

    @pl.when((b == _B - 1) & (c == _CHUNKS - 1))
    def _finish():
        total = acc_ref[3]
        valid = acc_ref[4]
        out_ref[0] = jnp.where(
            valid > 0.0, total / jnp.where(valid > 0.0, valid, 1.0), 0.0
        )


def kernel(predictions, labels, upper_region):
    out = pl.pallas_call(
        _loss_kernel,
        grid=(_B, _CHUNKS),
        in_specs=[
            pl.BlockSpec((1, _C, _RC, _W), lambda b, c: (b, 0, c, 0)),
            pl.BlockSpec((1, _RC, _W), lambda b, c: (b, c, 0)),
            pl.BlockSpec((1, _RC, _W), lambda b, c: (b, c, 0)),
        ],
        out_specs=pl.BlockSpec(memory_space=pltpu.SMEM),
        out_shape=jax.ShapeDtypeStruct((1,), jnp.float32),
        scratch_shapes=[pltpu.SMEM((8,), jnp.float32)],
    )(predictions, labels, upper_region)
    return out[0]


# speedup vs baseline: 5.8835x; 1.8094x over previous
import jax
import jax.numpy as jnp
from jax.experimental import pallas as pl
from jax.experimental.pallas import tpu as pltpu

_B, _C, _H, _W = 8, 2, 512, 512
_CHUNKS = 1
_RC = _H // _CHUNKS


def _loss_kernel(preds_ref, labels_ref, upper_ref, out_ref, acc_ref):
    b = pl.program_id(0)
    c = pl.program_id(1)

    l = labels_ref[0]
    u = upper_ref[0]
    p0 = preds_ref[0, 0]
    p1 = preds_ref[0, 1]

    neural = l == 255
    nonneural = (l == 0) & (u == 255)
    mask = neural | nonneural

    mx = jnp.maximum(p0, p1)
    mn = jnp.minimum(p0, p1)
    logz = mx + jnp.log1p(jnp.exp(mn - mx))
    ll = jnp.where(neural, p1, p0)
    val = jnp.where(mask, logz - ll, 0.0)

    s_part = jnp.sum(val)
    n1_part = jnp.sum(neural.astype(jnp.float32))
    n2_part = jnp.sum(nonneural.astype(jnp.float32))

    @pl.when(c == 0)
    def _reset_batch():
        acc_ref[0] = 0.0
        acc_ref[1] = 0.0
        acc_ref[2] = 0.0

    @pl.when((b == 0) & (c == 0))
    def _reset_total():
        acc_ref[3] = 0.0
        acc_ref[4] = 0.0

    acc_ref[0] += s_part
    acc_ref[1] += n1_part
    acc_ref[2] += n2_part

    @pl.when(c == _CHUNKS - 1)
    def _finish_batch():
        s = acc_ref[0]
        n1 = acc_ref[1]
        n2 = acc_ref[2]
        ok = (n1 > 0.0) & (n2 > 0.0)
        denom = n1 + n2
        contrib = s / jnp.where(denom > 0.0, denom, 1.0)
        acc_ref[3] += jnp.where(ok, contrib, 0.0)
        acc_ref[4] += jnp.where(ok, 1.0, 0.0)

    @pl.when((b == _B - 1) & (c == _CHUNKS - 1))
    def _finish():
        total = acc_ref[3]
        valid = acc_ref[4]
        out_ref[0] = jnp.where(
            valid > 0.0, total / jnp.where(valid > 0.0, valid, 1.0), 0.0
        )


def kernel(predictions, labels, upper_region):
    out = pl.pallas_call(
        _loss_kernel,
        grid=(_B, _CHUNKS),
        in_specs=[
            pl.BlockSpec((1, _C, _RC, _W), lambda b, c: (b, 0, c, 0)),
            pl.BlockSpec((1, _RC, _W), lambda b, c: (b, c, 0)),
            pl.BlockSpec((1, _RC, _W), lambda b, c: (b, c, 0)),
        ],
        out_specs=pl.BlockSpec(memory_space=pltpu.SMEM),
        out_shape=jax.ShapeDtypeStruct((1,), jnp.float32),
        scratch_shapes=[pltpu.SMEM((8,), jnp.float32)],
    )(predictions, labels, upper_region)
    return out[0]
